# baseline (speedup 1.0000x reference)
import functools

import jax
import jax.numpy as jnp
from jax import lax
from jax.experimental import pallas as pl
from jax.experimental.pallas import tpu as pltpu
from jax.experimental.pallas import tpu_sc as plsc

N = 10000
E = 320000
HID = 256
NUM_GRAPHS = 64
BN_EPS = 1e-5

CHUNK = 80
EDGES_PER_TILE = E // 16
NCHUNK = EDGES_PER_TILE // CHUNK
GROUP = 50
ROWS_PER_TILE = N // 16

BLK = 1000
NBLK = N // BLK


def _make_sc_scatter(dh, interpret=False):
    mesh = plsc.VectorSubcoreMesh(core_axis_name="c", subcore_axis_name="s")

    @functools.partial(
        pl.kernel,
        out_type=jax.ShapeDtypeStruct((N, 2 * dh), jnp.float32),
        mesh=mesh,
        interpret=interpret,
        compiler_params=pltpu.CompilerParams(use_tc_tiling_on_sc=False),
        scratch_types=[
            pltpu.VMEM((GROUP, CHUNK), jnp.int32),
            pltpu.VMEM((GROUP, CHUNK), jnp.int32),
            pltpu.VMEM((4, CHUNK, dh), jnp.float32),
            pltpu.VMEM_SHARED((N, dh), jnp.float32),
            [pltpu.SemaphoreType.DMA] * 4,
            [pltpu.SemaphoreType.DMA] * 4,
        ],
    )
    def sc_scatter(h2, srcx, dst2, zeros, agg, idxg, idxs, rows, acc,
                   gsems, ssems):
        c = lax.axis_index("c")
        s = lax.axis_index("s")
        rbase = s * ROWS_PER_TILE
        pltpu.sync_copy(zeros.at[pl.ds(rbase, ROWS_PER_TILE)],
                        acc.at[pl.ds(rbase, ROWS_PER_TILE)])
        ebase = s * NCHUNK
        plsc.subcore_barrier()

        def gather(j, b):
            return pltpu.make_async_copy(h2.at[idxg.at[j]], rows.at[b],
                                         gsems[b])

        def scatter(j, b):
            return pltpu.make_async_copy(rows.at[b], acc.at[idxs.at[j]],
                                         ssems[b])

        def group_body(g, carry):
            gb = ebase + g * GROUP
            pltpu.sync_copy(srcx.at[c, pl.ds(gb, GROUP)], idxg)
            pltpu.sync_copy(dst2.at[pl.ds(gb, GROUP)], idxs)
            gather(0, 0).start()
            gather(1, 1).start()

            def body(j4, carry2):
                for b in range(4):
                    j = j4 * 4 + b
                    gather(j, b).wait()
                    scatter(j, b).start(add=True)
                    b2 = (b + 2) % 4

                    @pl.when(j >= 2)
                    def _():
                        scatter(j - 2, b2).wait()

                    gather(j + 2, b2).start()
                return carry2

            lax.fori_loop(0, (GROUP - 2) // 4, body, 0)
            for t in range(2):
                j = GROUP - 2 + t
                gather(j, j % 4).wait()
                scatter(j, j % 4).start(add=True)
            for t in range(4):
                j = GROUP - 4 + t
                scatter(j, j % 4).wait()
            return carry

        lax.fori_loop(0, NCHUNK // GROUP, group_body, 0)
        plsc.subcore_barrier()
        pltpu.sync_copy(acc.at[pl.ds(rbase, ROWS_PER_TILE)],
                        agg.at[pl.ds(rbase, ROWS_PER_TILE), pl.ds(c * dh, dh)])

    return sc_scatter


_sc_scatter_cached = functools.cache(_make_sc_scatter)


def _xw1_body(h_ref, w1_ref, b1_ref, y_ref):
    y_ref[...] = jnp.dot(h_ref[...], w1_ref[...],
                         preferred_element_type=jnp.float32) + b1_ref[...]


def _xw1(h, w1, b1, interpret=False):
    d_in = h.shape[1]
    return pl.pallas_call(
        _xw1_body,
        grid=(NBLK,),
        in_specs=[
            pl.BlockSpec((BLK, d_in), lambda j: (j, 0)),
            pl.BlockSpec((d_in, HID), lambda j: (0, 0)),
            pl.BlockSpec((1, HID), lambda j: (0, 0)),
        ],
        out_specs=pl.BlockSpec((BLK, HID), lambda j: (j, 0)),
        out_shape=jax.ShapeDtypeStruct((N, HID), jnp.float32),
        compiler_params=pltpu.CompilerParams(
            dimension_semantics=("parallel",)),
        interpret=interpret,
    )(h, w1, b1)


def _mlp_body(y_ref, a_ref, w1h_ref, w2_ref, b2_ref, z_ref, st_ref):
    j = pl.program_id(0)
    z1 = jnp.maximum(
        y_ref[...] + jnp.dot(a_ref[...], w1h_ref[...],
                             preferred_element_type=jnp.float32),
        0.0)
    z = jnp.dot(z1, w2_ref[...], preferred_element_type=jnp.float32) + b2_ref[...]
    z_ref[...] = z
    s = jnp.sum(z, axis=0, keepdims=True)
    s2 = jnp.sum(z * z, axis=0, keepdims=True)
    upd = jnp.concatenate([s, s2, jnp.zeros((6, HID), jnp.float32)], axis=0)

    @pl.when(j == 0)
    def _():
        st_ref[...] = upd

    @pl.when(j > 0)
    def _():
        st_ref[...] += upd


def _mlp(yh, agg, w1, w2, b2, interpret=False):
    d_in = agg.shape[1]
    return pl.pallas_call(
        _mlp_body,
        grid=(NBLK,),
        in_specs=[
            pl.BlockSpec((BLK, HID), lambda j: (j, 0)),
            pl.BlockSpec((BLK, d_in), lambda j: (j, 0)),
            pl.BlockSpec((d_in, HID), lambda j: (0, 0)),
            pl.BlockSpec((HID, HID), lambda j: (0, 0)),
            pl.BlockSpec((1, HID), lambda j: (0, 0)),
        ],
        out_specs=[
            pl.BlockSpec((BLK, HID), lambda j: (j, 0)),
            pl.BlockSpec((8, HID), lambda j: (0, 0)),
        ],
        out_shape=[
            jax.ShapeDtypeStruct((N, HID), jnp.float32),
            jax.ShapeDtypeStruct((8, HID), jnp.float32),
        ],
        compiler_params=pltpu.CompilerParams(
            dimension_semantics=("arbitrary",)),
        interpret=interpret,
    )(yh, agg, w1, w2, b2)


def _bn_body(z_ref, st_ref, g_ref, bt_ref, h_ref):
    mean = st_ref[0:1, :] * (1.0 / N)
    ex2 = st_ref[1:2, :] * (1.0 / N)
    var = ex2 - mean * mean
    scale = g_ref[...] * lax.rsqrt(var + BN_EPS)
    h_ref[...] = jnp.maximum((z_ref[...] - mean) * scale + bt_ref[...], 0.0)


def _bn(z, st, gamma, beta, interpret=False):
    return pl.pallas_call(
        _bn_body,
        grid=(NBLK,),
        in_specs=[
            pl.BlockSpec((BLK, HID), lambda j: (j, 0)),
            pl.BlockSpec((8, HID), lambda j: (0, 0)),
            pl.BlockSpec((1, HID), lambda j: (0, 0)),
            pl.BlockSpec((1, HID), lambda j: (0, 0)),
        ],
        out_specs=pl.BlockSpec((BLK, HID), lambda j: (j, 0)),
        out_shape=jax.ShapeDtypeStruct((N, HID), jnp.float32),
        compiler_params=pltpu.CompilerParams(
            dimension_semantics=("parallel",)),
        interpret=interpret,
    )(z, st, gamma, beta)


def _bn_pool_body(z_ref, st_ref, g_ref, bt_ref, batch_ref, out_ref,
                  seg_acc, cnt_acc):
    j = pl.program_id(0)
    mean = st_ref[0:1, :] * (1.0 / N)
    ex2 = st_ref[1:2, :] * (1.0 / N)
    var = ex2 - mean * mean
    scale = g_ref[...] * lax.rsqrt(var + BN_EPS)
    h = jnp.maximum((z_ref[...] - mean) * scale + bt_ref[...], 0.0)
    gids = lax.broadcasted_iota(jnp.int32, (1, NUM_GRAPHS), 1)
    p = (batch_ref[...] == gids).astype(jnp.float32)
    dn = (((0,), (0,)), ((), ()))
    seg = lax.dot_general(p, h, dn, preferred_element_type=jnp.float32)
    cnt = lax.dot_general(p, jnp.ones((BLK, HID), jnp.float32), dn,
                          preferred_element_type=jnp.float32)

    @pl.when(j == 0)
    def _():
        seg_acc[...] = seg
        cnt_acc[...] = cnt

    @pl.when(j > 0)
    def _():
        seg_acc[...] += seg
        cnt_acc[...] += cnt

    @pl.when(j == NBLK - 1)
    def _():
        out_ref[...] = seg_acc[...] / jnp.maximum(cnt_acc[...], 1.0)


def _bn_pool(z, st, gamma, beta, batch2, interpret=False):
    return pl.pallas_call(
        _bn_pool_body,
        grid=(NBLK,),
        in_specs=[
            pl.BlockSpec((BLK, HID), lambda j: (j, 0)),
            pl.BlockSpec((8, HID), lambda j: (0, 0)),
            pl.BlockSpec((1, HID), lambda j: (0, 0)),
            pl.BlockSpec((1, HID), lambda j: (0, 0)),
            pl.BlockSpec((BLK, 1), lambda j: (j, 0)),
        ],
        out_specs=pl.BlockSpec((NUM_GRAPHS, HID), lambda j: (0, 0)),
        out_shape=jax.ShapeDtypeStruct((NUM_GRAPHS, HID), jnp.float32),
        scratch_shapes=[
            pltpu.VMEM((NUM_GRAPHS, HID), jnp.float32),
            pltpu.VMEM((NUM_GRAPHS, HID), jnp.float32),
        ],
        compiler_params=pltpu.CompilerParams(
            dimension_semantics=("arbitrary",)),
        interpret=interpret,
    )(z, st, gamma, beta, batch2)


def kernel(x, edge_index, batch, params):
    src = edge_index[0]
    dst = edge_index[1]
    srcx = jnp.stack([src * 2, src * 2 + 1]).reshape(2, E // CHUNK, CHUNK)
    dst2 = dst.reshape(E // CHUNK, CHUNK)
    batch2 = batch.reshape(N, 1)
    h = x
    out = None
    for i, p in enumerate(params):
        d_in = h.shape[1]
        dh = d_in // 2
        h2 = h.reshape(N * 2, dh)
        zeros = jnp.zeros((N, dh), jnp.float32)
        agg = _sc_scatter_cached(dh)(h2, srcx, dst2, zeros)
        yh = _xw1(h, p["W1"], p["b1"].reshape(1, HID))
        z, st = _mlp(yh, agg, p["W1"], p["W2"], p["b2"].reshape(1, HID))
        g = p["gamma"].reshape(1, HID)
        b = p["beta"].reshape(1, HID)
        if i + 1 < len(params):
            h = _bn(z, st, g, b)
        else:
            out = _bn_pool(z, st, g, b, batch2)
    return out


# speedup vs baseline: 1.0283x; 1.0283x over previous
import functools

import jax
import jax.numpy as jnp
from jax import lax
from jax.experimental import pallas as pl
from jax.experimental.pallas import tpu as pltpu
from jax.experimental.pallas import tpu_sc as plsc

N = 10000
E = 320000
HID = 256
NUM_GRAPHS = 64
BN_EPS = 1e-5

CHUNK = 80
EDGES_PER_TILE = E // 16
NCHUNK = EDGES_PER_TILE // CHUNK
GROUP = 50
ROWS_PER_TILE = N // 16

BLK = 1000
NBLK = N // BLK


def _make_sc_scatter(dh, interpret=False):
    mesh = plsc.VectorSubcoreMesh(core_axis_name="c", subcore_axis_name="s")

    @functools.partial(
        pl.kernel,
        out_type=jax.ShapeDtypeStruct((N, 2 * dh), jnp.float32),
        mesh=mesh,
        interpret=interpret,
        compiler_params=pltpu.CompilerParams(use_tc_tiling_on_sc=False),
        scratch_types=[
            pltpu.VMEM((GROUP, CHUNK), jnp.int32),
            pltpu.VMEM((GROUP, CHUNK), jnp.int32),
            pltpu.VMEM((4, CHUNK, dh), jnp.float32),
            pltpu.VMEM_SHARED((N, dh), jnp.float32),
            [pltpu.SemaphoreType.DMA] * 4,
            [pltpu.SemaphoreType.DMA] * 4,
        ],
    )
    def sc_scatter(h2, srcx, dst2, zeros, agg, idxg, idxs, rows, acc,
                   gsems, ssems):
        c = lax.axis_index("c")
        s = lax.axis_index("s")
        rbase = s * ROWS_PER_TILE
        pltpu.sync_copy(zeros.at[pl.ds(rbase, ROWS_PER_TILE)],
                        acc.at[pl.ds(rbase, ROWS_PER_TILE)])
        ebase = s * NCHUNK
        plsc.subcore_barrier()

        def gather(j, b):
            return pltpu.make_async_copy(h2.at[idxg.at[j]], rows.at[b],
                                         gsems[b])

        def scatter(j, b):
            return pltpu.make_async_copy(rows.at[b], acc.at[idxs.at[j]],
                                         ssems[b])

        def group_body(g, carry):
            gb = ebase + g * GROUP
            pltpu.sync_copy(srcx.at[c, pl.ds(gb, GROUP)], idxg)
            pltpu.sync_copy(dst2.at[pl.ds(gb, GROUP)], idxs)
            gather(0, 0).start()
            gather(1, 1).start()

            def body(j4, carry2):
                for b in range(4):
                    j = j4 * 4 + b
                    gather(j, b).wait()
                    scatter(j, b).start(add=True)
                    b2 = (b + 2) % 4

                    @pl.when(j >= 2)
                    def _():
                        scatter(j - 2, b2).wait()

                    gather(j + 2, b2).start()
                return carry2

            lax.fori_loop(0, (GROUP - 2) // 4, body, 0)
            for t in range(2):
                j = GROUP - 2 + t
                gather(j, j % 4).wait()
                scatter(j, j % 4).start(add=True)
            for t in range(4):
                j = GROUP - 4 + t
                scatter(j, j % 4).wait()
            return carry

        lax.fori_loop(0, NCHUNK // GROUP, group_body, 0)
        plsc.subcore_barrier()
        pltpu.sync_copy(acc.at[pl.ds(rbase, ROWS_PER_TILE)],
                        agg.at[pl.ds(rbase, ROWS_PER_TILE), pl.ds(c * dh, dh)])

    return sc_scatter


_sc_scatter_cached = functools.cache(_make_sc_scatter)


def _xw1_body(d_in, h2_ref, w1_ref, b1_ref, y_ref):
    dh = d_in // 2
    h3 = h2_ref[...].reshape(BLK, 2, dh)
    h = jnp.concatenate([h3[:, 0, :], h3[:, 1, :]], axis=1)
    y_ref[...] = jnp.dot(h, w1_ref[...],
                         preferred_element_type=jnp.float32) + b1_ref[...]


def _xw1(h2, w1, b1, interpret=False):
    d_in = 2 * h2.shape[1]
    dh = d_in // 2
    return pl.pallas_call(
        functools.partial(_xw1_body, d_in),
        grid=(NBLK,),
        in_specs=[
            pl.BlockSpec((2 * BLK, dh), lambda j: (j, 0)),
            pl.BlockSpec((d_in, HID), lambda j: (0, 0)),
            pl.BlockSpec((1, HID), lambda j: (0, 0)),
        ],
        out_specs=pl.BlockSpec((BLK, HID), lambda j: (j, 0)),
        out_shape=jax.ShapeDtypeStruct((N, HID), jnp.float32),
        compiler_params=pltpu.CompilerParams(
            dimension_semantics=("parallel",)),
        interpret=interpret,
    )(h2, w1, b1)


def _mlp_body(y_ref, a_ref, w1h_ref, w2_ref, b2_ref, z_ref, st_ref):
    j = pl.program_id(0)
    z1 = jnp.maximum(
        y_ref[...] + jnp.dot(a_ref[...], w1h_ref[...],
                             preferred_element_type=jnp.float32),
        0.0)
    z = jnp.dot(z1, w2_ref[...], preferred_element_type=jnp.float32) + b2_ref[...]
    z_ref[...] = z
    s = jnp.sum(z, axis=0, keepdims=True)
    s2 = jnp.sum(z * z, axis=0, keepdims=True)
    upd = jnp.concatenate([s, s2, jnp.zeros((6, HID), jnp.float32)], axis=0)

    @pl.when(j == 0)
    def _():
        st_ref[...] = upd

    @pl.when(j > 0)
    def _():
        st_ref[...] += upd


def _mlp(yh, agg, w1, w2, b2, interpret=False):
    d_in = agg.shape[1]
    return pl.pallas_call(
        _mlp_body,
        grid=(NBLK,),
        in_specs=[
            pl.BlockSpec((BLK, HID), lambda j: (j, 0)),
            pl.BlockSpec((BLK, d_in), lambda j: (j, 0)),
            pl.BlockSpec((d_in, HID), lambda j: (0, 0)),
            pl.BlockSpec((HID, HID), lambda j: (0, 0)),
            pl.BlockSpec((1, HID), lambda j: (0, 0)),
        ],
        out_specs=[
            pl.BlockSpec((BLK, HID), lambda j: (j, 0)),
            pl.BlockSpec((8, HID), lambda j: (0, 0)),
        ],
        out_shape=[
            jax.ShapeDtypeStruct((N, HID), jnp.float32),
            jax.ShapeDtypeStruct((8, HID), jnp.float32),
        ],
        compiler_params=pltpu.CompilerParams(
            dimension_semantics=("arbitrary",)),
        interpret=interpret,
    )(yh, agg, w1, w2, b2)


def _bn_body(z_ref, st_ref, g_ref, bt_ref, h2_ref):
    mean = st_ref[0:1, :] * (1.0 / N)
    ex2 = st_ref[1:2, :] * (1.0 / N)
    var = ex2 - mean * mean
    scale = g_ref[...] * lax.rsqrt(var + BN_EPS)
    h = jnp.maximum((z_ref[...] - mean) * scale + bt_ref[...], 0.0)
    dh = HID // 2
    hs = jnp.stack([h[:, :dh], h[:, dh:]], axis=1)
    h2_ref[...] = hs.reshape(2 * BLK, dh)


def _bn(z, st, gamma, beta, interpret=False):
    return pl.pallas_call(
        _bn_body,
        grid=(NBLK,),
        in_specs=[
            pl.BlockSpec((BLK, HID), lambda j: (j, 0)),
            pl.BlockSpec((8, HID), lambda j: (0, 0)),
            pl.BlockSpec((1, HID), lambda j: (0, 0)),
            pl.BlockSpec((1, HID), lambda j: (0, 0)),
        ],
        out_specs=pl.BlockSpec((2 * BLK, HID // 2), lambda j: (j, 0)),
        out_shape=jax.ShapeDtypeStruct((2 * N, HID // 2), jnp.float32),
        compiler_params=pltpu.CompilerParams(
            dimension_semantics=("parallel",)),
        interpret=interpret,
    )(z, st, gamma, beta)


def _bn_pool_body(z_ref, st_ref, g_ref, bt_ref, batch_ref, out_ref,
                  seg_acc, cnt_acc):
    j = pl.program_id(0)
    mean = st_ref[0:1, :] * (1.0 / N)
    ex2 = st_ref[1:2, :] * (1.0 / N)
    var = ex2 - mean * mean
    scale = g_ref[...] * lax.rsqrt(var + BN_EPS)
    h = jnp.maximum((z_ref[...] - mean) * scale + bt_ref[...], 0.0)
    gids = lax.broadcasted_iota(jnp.int32, (1, NUM_GRAPHS), 1)
    p = (batch_ref[...] == gids).astype(jnp.float32)
    dn = (((0,), (0,)), ((), ()))
    seg = lax.dot_general(p, h, dn, preferred_element_type=jnp.float32)
    cnt = lax.dot_general(p, jnp.ones((BLK, HID), jnp.float32), dn,
                          preferred_element_type=jnp.float32)

    @pl.when(j == 0)
    def _():
        seg_acc[...] = seg
        cnt_acc[...] = cnt

    @pl.when(j > 0)
    def _():
        seg_acc[...] += seg
        cnt_acc[...] += cnt

    @pl.when(j == NBLK - 1)
    def _():
        out_ref[...] = seg_acc[...] / jnp.maximum(cnt_acc[...], 1.0)


def _bn_pool(z, st, gamma, beta, batch2, interpret=False):
    return pl.pallas_call(
        _bn_pool_body,
        grid=(NBLK,),
        in_specs=[
            pl.BlockSpec((BLK, HID), lambda j: (j, 0)),
            pl.BlockSpec((8, HID), lambda j: (0, 0)),
            pl.BlockSpec((1, HID), lambda j: (0, 0)),
            pl.BlockSpec((1, HID), lambda j: (0, 0)),
            pl.BlockSpec((BLK, 1), lambda j: (j, 0)),
        ],
        out_specs=pl.BlockSpec((NUM_GRAPHS, HID), lambda j: (0, 0)),
        out_shape=jax.ShapeDtypeStruct((NUM_GRAPHS, HID), jnp.float32),
        scratch_shapes=[
            pltpu.VMEM((NUM_GRAPHS, HID), jnp.float32),
            pltpu.VMEM((NUM_GRAPHS, HID), jnp.float32),
        ],
        compiler_params=pltpu.CompilerParams(
            dimension_semantics=("arbitrary",)),
        interpret=interpret,
    )(z, st, gamma, beta, batch2)


def kernel(x, edge_index, batch, params):
    src = edge_index[0]
    dst = edge_index[1]
    srcx = jnp.stack([src * 2, src * 2 + 1]).reshape(2, E // CHUNK, CHUNK)
    dst2 = dst.reshape(E // CHUNK, CHUNK)
    batch2 = batch.reshape(N, 1)
    h2 = x.reshape(N * 2, x.shape[1] // 2)
    out = None
    for i, p in enumerate(params):
        dh = h2.shape[1]
        zeros = jnp.zeros((N, dh), jnp.float32)
        agg = _sc_scatter_cached(dh)(h2, srcx, dst2, zeros)
        yh = _xw1(h2, p["W1"], p["b1"].reshape(1, HID))
        z, st = _mlp(yh, agg, p["W1"], p["W2"], p["b2"].reshape(1, HID))
        g = p["gamma"].reshape(1, HID)
        b = p["beta"].reshape(1, HID)
        if i + 1 < len(params):
            h2 = _bn(z, st, g, b)
        else:
            out = _bn_pool(z, st, g, b, batch2)
    return out


# speedup vs baseline: 1.0844x; 1.0546x over previous
import functools

import jax
import jax.numpy as jnp
from jax import lax
from jax.experimental import pallas as pl
from jax.experimental.pallas import tpu as pltpu
from jax.experimental.pallas import tpu_sc as plsc

N = 10000
E = 320000
HID = 256
NUM_GRAPHS = 64
BN_EPS = 1e-5

CHUNK = 80
EDGES_PER_TILE = E // 16
NCHUNK = EDGES_PER_TILE // CHUNK
GROUP = 50
ROWS_PER_TILE = N // 16

BLK = 1000
NBLK = N // BLK


def _make_sc_scatter(dh, interpret=False):
    mesh = plsc.VectorSubcoreMesh(core_axis_name="c", subcore_axis_name="s")
    out_shape = (N, 2 * dh) if dh == 64 else (2 * N, dh)

    @functools.partial(
        pl.kernel,
        out_type=jax.ShapeDtypeStruct(out_shape, jnp.float32),
        mesh=mesh,
        interpret=interpret,
        compiler_params=pltpu.CompilerParams(use_tc_tiling_on_sc=False),
        scratch_types=[
            pltpu.VMEM((GROUP, CHUNK), jnp.int32),
            pltpu.VMEM((GROUP, CHUNK), jnp.int32),
            pltpu.VMEM((4, CHUNK, dh), jnp.float32),
            pltpu.VMEM_SHARED((N, dh), jnp.float32),
            [pltpu.SemaphoreType.DMA] * 4,
            [pltpu.SemaphoreType.DMA] * 4,
        ],
    )
    def sc_scatter(ha, hb, src2, dst2, zeros, agg, idxg, idxs, rows, acc,
                   gsems, ssems):
        c = lax.axis_index("c")
        s = lax.axis_index("s")
        rbase = s * ROWS_PER_TILE
        pltpu.sync_copy(zeros.at[pl.ds(rbase, ROWS_PER_TILE)],
                        acc.at[pl.ds(rbase, ROWS_PER_TILE)])
        ebase = s * NCHUNK
        plsc.subcore_barrier()

        def gather_start(j, b):
            @pl.when(c == 0)
            def _():
                pltpu.make_async_copy(ha.at[idxg.at[j]], rows.at[b],
                                      gsems[b]).start()

            @pl.when(c == 1)
            def _():
                pltpu.make_async_copy(hb.at[idxg.at[j]], rows.at[b],
                                      gsems[b]).start()

        def gather_wait(j, b):
            pltpu.make_async_copy(ha.at[idxg.at[j]], rows.at[b],
                                  gsems[b]).wait()

        def scatter(j, b):
            return pltpu.make_async_copy(rows.at[b], acc.at[idxs.at[j]],
                                         ssems[b])

        def group_body(g, carry):
            gb = ebase + g * GROUP
            pltpu.sync_copy(src2.at[pl.ds(gb, GROUP)], idxg)
            pltpu.sync_copy(dst2.at[pl.ds(gb, GROUP)], idxs)
            gather_start(0, 0)
            gather_start(1, 1)

            def body(j4, carry2):
                for b in range(4):
                    j = j4 * 4 + b
                    gather_wait(j, b)
                    scatter(j, b).start(add=True)
                    b2 = (b + 2) % 4

                    @pl.when(j >= 2)
                    def _():
                        scatter(j - 2, b2).wait()

                    gather_start(j + 2, b2)
                return carry2

            lax.fori_loop(0, (GROUP - 2) // 4, body, 0)
            for t in range(2):
                j = GROUP - 2 + t
                gather_wait(j, j % 4)
                scatter(j, j % 4).start(add=True)
            for t in range(4):
                j = GROUP - 4 + t
                scatter(j, j % 4).wait()
            return carry

        lax.fori_loop(0, NCHUNK // GROUP, group_body, 0)
        plsc.subcore_barrier()
        if dh == 64:
            dst = agg.at[pl.ds(rbase, ROWS_PER_TILE), pl.ds(c * dh, dh)]
        else:
            dst = agg.at[pl.ds(c * N + rbase, ROWS_PER_TILE)]
        pltpu.sync_copy(acc.at[pl.ds(rbase, ROWS_PER_TILE)], dst)

    return sc_scatter


_sc_scatter_cached = functools.cache(_make_sc_scatter)


def _xw1_flat_body(h_ref, w1_ref, b1_ref, y_ref):
    y_ref[...] = jnp.dot(h_ref[...], w1_ref[...],
                         preferred_element_type=jnp.float32) + b1_ref[...]


def _xw1_flat(h, w1, b1, interpret=False):
    d_in = h.shape[1]
    return pl.pallas_call(
        _xw1_flat_body,
        grid=(NBLK,),
        in_specs=[
            pl.BlockSpec((BLK, d_in), lambda j: (j, 0)),
            pl.BlockSpec((d_in, HID), lambda j: (0, 0)),
            pl.BlockSpec((1, HID), lambda j: (0, 0)),
        ],
        out_specs=pl.BlockSpec((BLK, HID), lambda j: (j, 0)),
        out_shape=jax.ShapeDtypeStruct((N, HID), jnp.float32),
        compiler_params=pltpu.CompilerParams(
            dimension_semantics=("parallel",)),
        interpret=interpret,
    )(h, w1, b1)


def _xw1_halves_body(ha_ref, hb_ref, w1_ref, b1_ref, y_ref):
    h = jnp.concatenate([ha_ref[...], hb_ref[...]], axis=1)
    y_ref[...] = jnp.dot(h, w1_ref[...],
                         preferred_element_type=jnp.float32) + b1_ref[...]


def _xw1_halves(ha, hb, w1, b1, interpret=False):
    dh = ha.shape[1]
    return pl.pallas_call(
        _xw1_halves_body,
        grid=(NBLK,),
        in_specs=[
            pl.BlockSpec((BLK, dh), lambda j: (j, 0)),
            pl.BlockSpec((BLK, dh), lambda j: (j, 0)),
            pl.BlockSpec((2 * dh, HID), lambda j: (0, 0)),
            pl.BlockSpec((1, HID), lambda j: (0, 0)),
        ],
        out_specs=pl.BlockSpec((BLK, HID), lambda j: (j, 0)),
        out_shape=jax.ShapeDtypeStruct((N, HID), jnp.float32),
        compiler_params=pltpu.CompilerParams(
            dimension_semantics=("parallel",)),
        interpret=interpret,
    )(ha, hb, w1, b1)


def _mlp_body(stacked, y_ref, al_ref, ar_ref, w1_ref, w2_ref, b2_ref,
              z_ref, st_ref):
    j = pl.program_id(0)
    if stacked:
        a = jnp.concatenate([al_ref[...], ar_ref[...]], axis=1)
    else:
        a = al_ref[...]
    z1 = jnp.maximum(
        y_ref[...] + jnp.dot(a, w1_ref[...],
                             preferred_element_type=jnp.float32),
        0.0)
    z = jnp.dot(z1, w2_ref[...], preferred_element_type=jnp.float32) + b2_ref[...]
    z_ref[...] = z
    s = jnp.sum(z, axis=0, keepdims=True)
    s2 = jnp.sum(z * z, axis=0, keepdims=True)
    upd = jnp.concatenate([s, s2, jnp.zeros((6, HID), jnp.float32)], axis=0)

    @pl.when(j == 0)
    def _():
        st_ref[...] = upd

    @pl.when(j > 0)
    def _():
        st_ref[...] += upd


def _mlp(yh, agg, w1, w2, b2, interpret=False):
    stacked = agg.shape[0] == 2 * N
    d_in = w1.shape[0]
    if stacked:
        al_spec = pl.BlockSpec((BLK, HID // 2), lambda j: (j, 0))
        ar_spec = pl.BlockSpec((BLK, HID // 2), lambda j: (j + NBLK, 0))
    else:
        al_spec = pl.BlockSpec((BLK, d_in), lambda j: (j, 0))
        ar_spec = pl.BlockSpec((BLK, d_in), lambda j: (j, 0))
    return pl.pallas_call(
        functools.partial(_mlp_body, stacked),
        grid=(NBLK,),
        in_specs=[
            pl.BlockSpec((BLK, HID), lambda j: (j, 0)),
            al_spec,
            ar_spec,
            pl.BlockSpec((d_in, HID), lambda j: (0, 0)),
            pl.BlockSpec((HID, HID), lambda j: (0, 0)),
            pl.BlockSpec((1, HID), lambda j: (0, 0)),
        ],
        out_specs=[
            pl.BlockSpec((BLK, HID), lambda j: (j, 0)),
            pl.BlockSpec((8, HID), lambda j: (0, 0)),
        ],
        out_shape=[
            jax.ShapeDtypeStruct((N, HID), jnp.float32),
            jax.ShapeDtypeStruct((8, HID), jnp.float32),
        ],
        compiler_params=pltpu.CompilerParams(
            dimension_semantics=("arbitrary",)),
        interpret=interpret,
    )(yh, agg, agg, w1, w2, b2)


def _bn_body(z_ref, st_ref, g_ref, bt_ref, ha_ref, hb_ref):
    mean = st_ref[0:1, :] * (1.0 / N)
    ex2 = st_ref[1:2, :] * (1.0 / N)
    var = ex2 - mean * mean
    scale = g_ref[...] * lax.rsqrt(var + BN_EPS)
    h = jnp.maximum((z_ref[...] - mean) * scale + bt_ref[...], 0.0)
    dh = HID // 2
    ha_ref[...] = h[:, :dh]
    hb_ref[...] = h[:, dh:]


def _bn(z, st, gamma, beta, interpret=False):
    dh = HID // 2
    return pl.pallas_call(
        _bn_body,
        grid=(NBLK,),
        in_specs=[
            pl.BlockSpec((BLK, HID), lambda j: (j, 0)),
            pl.BlockSpec((8, HID), lambda j: (0, 0)),
            pl.BlockSpec((1, HID), lambda j: (0, 0)),
            pl.BlockSpec((1, HID), lambda j: (0, 0)),
        ],
        out_specs=[
            pl.BlockSpec((BLK, dh), lambda j: (j, 0)),
            pl.BlockSpec((BLK, dh), lambda j: (j, 0)),
        ],
        out_shape=[
            jax.ShapeDtypeStruct((N, dh), jnp.float32),
            jax.ShapeDtypeStruct((N, dh), jnp.float32),
        ],
        compiler_params=pltpu.CompilerParams(
            dimension_semantics=("parallel",)),
        interpret=interpret,
    )(z, st, gamma, beta)


def _bn_pool_body(z_ref, st_ref, g_ref, bt_ref, batch_ref, out_ref,
                  seg_acc, cnt_acc):
    j = pl.program_id(0)
    mean = st_ref[0:1, :] * (1.0 / N)
    ex2 = st_ref[1:2, :] * (1.0 / N)
    var = ex2 - mean * mean
    scale = g_ref[...] * lax.rsqrt(var + BN_EPS)
    h = jnp.maximum((z_ref[...] - mean) * scale + bt_ref[...], 0.0)
    gids = lax.broadcasted_iota(jnp.int32, (1, NUM_GRAPHS), 1)
    p = (batch_ref[...] == gids).astype(jnp.float32)
    dn = (((0,), (0,)), ((), ()))
    seg = lax.dot_general(p, h, dn, preferred_element_type=jnp.float32)
    cnt = lax.dot_general(p, jnp.ones((BLK, HID), jnp.float32), dn,
                          preferred_element_type=jnp.float32)

    @pl.when(j == 0)
    def _():
        seg_acc[...] = seg
        cnt_acc[...] = cnt

    @pl.when(j > 0)
    def _():
        seg_acc[...] += seg
        cnt_acc[...] += cnt

    @pl.when(j == NBLK - 1)
    def _():
        out_ref[...] = seg_acc[...] / jnp.maximum(cnt_acc[...], 1.0)


def _bn_pool(z, st, gamma, beta, batch2, interpret=False):
    return pl.pallas_call(
        _bn_pool_body,
        grid=(NBLK,),
        in_specs=[
            pl.BlockSpec((BLK, HID), lambda j: (j, 0)),
            pl.BlockSpec((8, HID), lambda j: (0, 0)),
            pl.BlockSpec((1, HID), lambda j: (0, 0)),
            pl.BlockSpec((1, HID), lambda j: (0, 0)),
            pl.BlockSpec((BLK, 1), lambda j: (j, 0)),
        ],
        out_specs=pl.BlockSpec((NUM_GRAPHS, HID), lambda j: (0, 0)),
        out_shape=jax.ShapeDtypeStruct((NUM_GRAPHS, HID), jnp.float32),
        scratch_shapes=[
            pltpu.VMEM((NUM_GRAPHS, HID), jnp.float32),
            pltpu.VMEM((NUM_GRAPHS, HID), jnp.float32),
        ],
        compiler_params=pltpu.CompilerParams(
            dimension_semantics=("arbitrary",)),
        interpret=interpret,
    )(z, st, gamma, beta, batch2)


def kernel(x, edge_index, batch, params):
    src = edge_index[0]
    dst = edge_index[1]
    src2 = src.reshape(E // CHUNK, CHUNK)
    dst2 = dst.reshape(E // CHUNK, CHUNK)
    batch2 = batch.reshape(N, 1)
    d0 = x.shape[1]
    ha, hb = x[:, :d0 // 2], x[:, d0 // 2:]
    out = None
    for i, p in enumerate(params):
        dh = ha.shape[1]
        zeros = jnp.zeros((N, dh), jnp.float32)
        agg = _sc_scatter_cached(dh)(ha, hb, src2, dst2, zeros)
        if i == 0:
            yh = _xw1_flat(x, p["W1"], p["b1"].reshape(1, HID))
        else:
            yh = _xw1_halves(ha, hb, p["W1"], p["b1"].reshape(1, HID))
        z, st = _mlp(yh, agg, p["W1"], p["W2"], p["b2"].reshape(1, HID))
        g = p["gamma"].reshape(1, HID)
        b = p["beta"].reshape(1, HID)
        if i + 1 < len(params):
            ha, hb = _bn(z, st, g, b)
        else:
            out = _bn_pool(z, st, g, b, batch2)
    return out
